# baseline (speedup 1.0000x reference)
import jax
import jax.numpy as jnp
from jax.experimental import pallas as pl
from jax.experimental.pallas import tpu as pltpu

_N = 10000
_D = 128
_BM = 400


def _layer_kernel(adj_ref, x_ref, w_ref, b_ref, m_ref, out_ref):
    a = adj_ref[...].astype(jnp.bfloat16)
    xb = x_ref[...].astype(jnp.bfloat16)
    h = jax.lax.dot(a, xb, preferred_element_type=jnp.float32)
    y = jax.lax.dot(h, w_ref[...], preferred_element_type=jnp.float32)
    y = (y + b_ref[...]) * m_ref[...]
    out_ref[...] = jnp.maximum(y, 0.0)


def _layer(adj, x, w, b2d, m2d):
    return pl.pallas_call(
        _layer_kernel,
        grid=(_N // _BM,),
        in_specs=[
            pl.BlockSpec((_BM, _N), lambda i: (i, 0)),
            pl.BlockSpec((_N, _D), lambda i: (0, 0)),
            pl.BlockSpec((_D, _D), lambda i: (0, 0)),
            pl.BlockSpec((1, _D), lambda i: (0, 0)),
            pl.BlockSpec((_BM, 1), lambda i: (i, 0)),
        ],
        out_specs=pl.BlockSpec((_BM, _D), lambda i: (i, 0)),
        out_shape=jax.ShapeDtypeStruct((_N, _D), jnp.float32),
        compiler_params=pltpu.CompilerParams(
            dimension_semantics=("parallel",),
        ),
    )(adj, x, w, b2d, m2d)


def kernel(x, adj, mask, W0, b0, W1, b1):
    m2d = mask.astype(jnp.float32)[:, None]
    y = _layer(adj, x, W0, b0[None, :], m2d)
    y = _layer(adj, y, W1, b1[None, :], m2d)
    return y


# speedup vs baseline: 1.0021x; 1.0021x over previous
import jax
import jax.numpy as jnp
from jax.experimental import pallas as pl
from jax.experimental.pallas import tpu as pltpu

_N = 10000
_D = 128
_BM = 400


def _layer_kernel(adj_ref, x_ref, w_ref, b_ref, m_ref, out_ref):
    h = jax.lax.dot(adj_ref[...], x_ref[...], preferred_element_type=jnp.float32)
    y = jax.lax.dot(h, w_ref[...], preferred_element_type=jnp.float32)
    y = (y + b_ref[...]) * m_ref[...]
    out_ref[...] = jnp.maximum(y, 0.0)


def _layer(adj, x, w, b2d, m2d):
    return pl.pallas_call(
        _layer_kernel,
        grid=(_N // _BM,),
        in_specs=[
            pl.BlockSpec((_BM, _N), lambda i: (i, 0)),
            pl.BlockSpec((_N, _D), lambda i: (0, 0)),
            pl.BlockSpec((_D, _D), lambda i: (0, 0)),
            pl.BlockSpec((1, _D), lambda i: (0, 0)),
            pl.BlockSpec((_BM, 1), lambda i: (i, 0)),
        ],
        out_specs=pl.BlockSpec((_BM, _D), lambda i: (i, 0)),
        out_shape=jax.ShapeDtypeStruct((_N, _D), jnp.float32),
        compiler_params=pltpu.CompilerParams(
            dimension_semantics=("parallel",),
        ),
    )(adj, x, w, b2d, m2d)


def kernel(x, adj, mask, W0, b0, W1, b1):
    m2d = mask.astype(jnp.float32)[:, None]
    y = _layer(adj, x, W0, b0[None, :], m2d)
    y = _layer(adj, y, W1, b1[None, :], m2d)
    return y
